# Initial kernel scaffold; baseline (speedup 1.0000x reference)
import jax
import jax.numpy as jnp
from jax.experimental import pallas as pl


def kernel(all_rois, gt_boxes, num_boxes):
    raise NotImplementedError("write your pallas kernel here")


# speedup vs baseline: 1.0449x; 1.0449x over previous
import functools

import jax
import jax.numpy as jnp
from jax import lax
from jax.experimental import pallas as pl
from jax.experimental.pallas import tpu as pltpu

_NCLASSES = 21
_FG_ROIS = 32
_BG_ROIS = 96
_ROIS = _FG_ROIS + _BG_ROIS
_FG_THRESH = 0.5
_BG_HI = 0.5
_BG_LO = 0.1
_STDS = (0.1, 0.1, 0.2, 0.2)


def _body(m_total, k_gt, nb_ref, rois_ref, gtt_ref,
          out_rois, out_t, out_in, out_out,
          mo_ref, bk_ref, keep_ref):
    b = pl.program_id(0)
    rows = rois_ref.shape[2]
    nb = nb_ref[0, 0, 0]

    rx1 = rois_ref[0, 0]
    ry1 = rois_ref[0, 1]
    rx2 = rois_ref[0, 2]
    ry2 = rois_ref[0, 3]
    rarea = (rx2 - rx1 + 1.0) * (ry2 - ry1 + 1.0)

    mo = jnp.full((rows, 128), -2.0, jnp.float32)
    bk = jnp.zeros((rows, 128), jnp.int32)
    for k in range(k_gt):
        gx1 = gtt_ref[0, 0, k]
        gy1 = gtt_ref[0, 1, k]
        gx2 = gtt_ref[0, 2, k]
        gy2 = gtt_ref[0, 3, k]
        garea = (gx2 - gx1 + 1.0) * (gy2 - gy1 + 1.0)
        iw = jnp.clip(jnp.minimum(rx2, gx2) - jnp.maximum(rx1, gx1) + 1.0, 0.0)
        ih = jnp.clip(jnp.minimum(ry2, gy2) - jnp.maximum(ry1, gy1) + 1.0, 0.0)
        inter = iw * ih
        iou = inter / (rarea + garea - inter)
        score = jnp.where(k < nb, iou, -1.0)
        upd = score > mo
        mo = jnp.where(upd, score, mo)
        bk = jnp.where(upd, k, bk)
    mo_ref[...] = mo
    bk_ref[...] = bk

    lin = (lax.broadcasted_iota(jnp.int32, (rows, 128), 0) * 128
           + lax.broadcasted_iota(jnp.int32, (rows, 128), 1))
    col_ok = lin < m_total
    fgs = jnp.where(col_ok, jnp.where(mo >= _FG_THRESH, mo, -1.0), -2.0)
    bgs = jnp.where(col_ok,
                    jnp.where((mo < _BG_HI) & (mo >= _BG_LO), 1.0 - mo, -1.0),
                    -2.0)

    big = jnp.int32(rows * 128)

    def extract(off):
        def step(i, s):
            m = jnp.max(s)
            idx = jnp.min(jnp.where(s == m, lin, big))
            keep_ref[off + i] = idx
            return jnp.where(lin == idx, -3.0, s)
        return step

    lax.fori_loop(0, _FG_ROIS, extract(0), fgs)
    lax.fori_loop(0, _BG_ROIS, extract(_FG_ROIS), bgs)

    bf = b.astype(jnp.float32)
    colio = lax.broadcasted_iota(jnp.int32, (1, 4 * _NCLASSES), 1)
    cio8 = lax.broadcasted_iota(jnp.int32, (1, 8), 1)
    iota_k = lax.broadcasted_iota(jnp.int32, (1, k_gt), 1)
    rowsel = lax.broadcasted_iota(jnp.int32, (_ROIS, 1), 0)

    gxv1 = gtt_ref[0, pl.ds(0, 1), :]
    gyv1 = gtt_ref[0, pl.ds(1, 1), :]
    gxv2 = gtt_ref[0, pl.ds(2, 1), :]
    gyv2 = gtt_ref[0, pl.ds(3, 1), :]
    gclsv = gtt_ref[0, pl.ds(4, 1), :]

    mo2 = mo_ref[...]
    bk2 = bk_ref[...]

    def emit(i, carry):
        acc_rois, acc_t, acc_w = carry
        idx = keep_ref[i]
        m1 = lin == idx
        ex1 = jnp.sum(jnp.where(m1, rx1, 0.0))
        ey1 = jnp.sum(jnp.where(m1, ry1, 0.0))
        ex2 = jnp.sum(jnp.where(m1, rx2, 0.0))
        ey2 = jnp.sum(jnp.where(m1, ry2, 0.0))
        mo_s = jnp.sum(jnp.where(m1, mo2, 0.0))
        ga = jnp.sum(jnp.where(m1, bk2, 0))
        m2 = iota_k == ga
        gx1 = jnp.sum(jnp.where(m2, gxv1, 0.0))
        gy1 = jnp.sum(jnp.where(m2, gyv1, 0.0))
        gx2 = jnp.sum(jnp.where(m2, gxv2, 0.0))
        gy2 = jnp.sum(jnp.where(m2, gyv2, 0.0))
        gcls = jnp.sum(jnp.where(m2, gclsv, 0.0))

        ew = ex2 - ex1 + 1.0
        eh = ey2 - ey1 + 1.0
        ecx = ex1 + 0.5 * ew
        ecy = ey1 + 0.5 * eh
        gw = gx2 - gx1 + 1.0
        gh = gy2 - gy1 + 1.0
        gcx = gx1 + 0.5 * gw
        gcy = gy1 + 0.5 * gh
        dx = ((gcx - ecx) / ew) / _STDS[0]
        dy = ((gcy - ecy) / eh) / _STDS[1]
        dw = jnp.log(gw / ew) / _STDS[2]
        dh = jnp.log(gh / eh) / _STDS[3]

        fg = mo_s >= _FG_THRESH
        label = jnp.where(fg, gcls, 0.0)

        row = jnp.where(
            cio8 == 0, bf,
            jnp.where(cio8 == 1, ex1,
                      jnp.where(cio8 == 2, ey1,
                                jnp.where(cio8 == 3, ex2,
                                          jnp.where(cio8 == 4, ey2,
                                                    jnp.where(cio8 == 5, label,
                                                              0.0))))))

        base = label.astype(jnp.int32) * 4
        off = colio - base
        inr = (off >= 0) & (off < 4) & fg
        tval = jnp.where(off == 0, dx,
                         jnp.where(off == 1, dy,
                                   jnp.where(off == 2, dw, dh)))
        zero = jnp.zeros((1, 4 * _NCLASSES), jnp.float32)
        trow = jnp.where(inr, tval, zero)
        wrow = jnp.where(inr, 1.0, zero)

        here = rowsel == i
        acc_rois = jnp.where(here, row, acc_rois)
        acc_t = jnp.where(here, trow, acc_t)
        acc_w = jnp.where(here, wrow, acc_w)
        return acc_rois, acc_t, acc_w

    acc0 = (jnp.zeros((_ROIS, 8), jnp.float32),
            jnp.zeros((_ROIS, 4 * _NCLASSES), jnp.float32),
            jnp.zeros((_ROIS, 4 * _NCLASSES), jnp.float32))
    acc_rois, acc_t, acc_w = lax.fori_loop(0, _ROIS, emit, acc0)
    out_rois[0] = acc_rois
    out_t[0] = acc_t
    out_in[0] = acc_w
    out_out[0] = acc_w


@jax.jit
def kernel(all_rois, gt_boxes, num_boxes):
    B, N, _ = all_rois.shape
    K = gt_boxes.shape[1]
    M = N + K
    MP = ((M + 127) // 128) * 128
    rows = MP // 128

    coords = jnp.concatenate([all_rois[:, :, 1:5], gt_boxes[:, :, :4]], axis=1)
    coords = jnp.pad(coords, ((0, 0), (0, MP - M), (0, 0)))
    rois_t = coords.transpose(0, 2, 1).reshape(B, 4, rows, 128)
    gtt = gt_boxes.transpose(0, 2, 1)
    nb = num_boxes.astype(jnp.int32).reshape(B, 1, 1)

    out_shapes = (
        jax.ShapeDtypeStruct((B, _ROIS, 8), jnp.float32),
        jax.ShapeDtypeStruct((B, _ROIS, 4 * _NCLASSES), jnp.float32),
        jax.ShapeDtypeStruct((B, _ROIS, 4 * _NCLASSES), jnp.float32),
        jax.ShapeDtypeStruct((B, _ROIS, 4 * _NCLASSES), jnp.float32),
    )
    grid_spec = pltpu.PrefetchScalarGridSpec(
        num_scalar_prefetch=0,
        grid=(B,),
        in_specs=[
            pl.BlockSpec((1, 1, 1), lambda b: (b, 0, 0),
                         memory_space=pltpu.SMEM),
            pl.BlockSpec((1, 4, rows, 128), lambda b: (b, 0, 0, 0)),
            pl.BlockSpec((1, 5, K), lambda b: (b, 0, 0)),
        ],
        out_specs=[
            pl.BlockSpec((1, _ROIS, 8), lambda b: (b, 0, 0)),
            pl.BlockSpec((1, _ROIS, 4 * _NCLASSES), lambda b: (b, 0, 0)),
            pl.BlockSpec((1, _ROIS, 4 * _NCLASSES), lambda b: (b, 0, 0)),
            pl.BlockSpec((1, _ROIS, 4 * _NCLASSES), lambda b: (b, 0, 0)),
        ],
        scratch_shapes=[
            pltpu.VMEM((rows, 128), jnp.float32),
            pltpu.VMEM((rows, 128), jnp.int32),
            pltpu.SMEM((_ROIS,), jnp.int32),
        ],
    )
    fn = pl.pallas_call(
        functools.partial(_body, M, K),
        grid_spec=grid_spec,
        out_shape=out_shapes,
    )
    comb, tgt, win, wout = fn(nb, rois_t, gtt)
    return comb[:, :, :5], comb[:, :, 5], tgt, win, wout


# speedup vs baseline: 1.4991x; 1.4347x over previous
import functools

import jax
import jax.numpy as jnp
from jax import lax
from jax.experimental import pallas as pl
from jax.experimental.pallas import tpu as pltpu

_NCLASSES = 21
_FG_ROIS = 32
_BG_ROIS = 96
_ROIS = _FG_ROIS + _BG_ROIS
_FG_THRESH = 0.5
_BG_HI = 0.5
_BG_LO = 0.1
_STDS = (0.1, 0.1, 0.2, 0.2)


def _body(m_total, k_gt, batch, nb_ref, rois_ref, gtt_ref,
          out_rois, out_t, out_in, out_out,
          mo_ref, bk_ref, keep_ref):
    rows = rois_ref.shape[2]
    lin = (lax.broadcasted_iota(jnp.int32, (rows, 128), 0) * 128
           + lax.broadcasted_iota(jnp.int32, (rows, 128), 1))
    col_ok = lin < m_total
    big = jnp.int32(rows * 128)

    fgs = []
    bgs = []
    for b in range(batch):
        nb = nb_ref[b, 0, 0]
        rx1 = rois_ref[b, 0]
        ry1 = rois_ref[b, 1]
        rx2 = rois_ref[b, 2]
        ry2 = rois_ref[b, 3]
        rarea = (rx2 - rx1 + 1.0) * (ry2 - ry1 + 1.0)
        mo = jnp.full((rows, 128), -2.0, jnp.float32)
        bk = jnp.zeros((rows, 128), jnp.int32)
        for k in range(k_gt):
            gx1 = gtt_ref[b, 0, k]
            gy1 = gtt_ref[b, 1, k]
            gx2 = gtt_ref[b, 2, k]
            gy2 = gtt_ref[b, 3, k]
            garea = (gx2 - gx1 + 1.0) * (gy2 - gy1 + 1.0)
            iw = jnp.clip(jnp.minimum(rx2, gx2) - jnp.maximum(rx1, gx1) + 1.0,
                          0.0)
            ih = jnp.clip(jnp.minimum(ry2, gy2) - jnp.maximum(ry1, gy1) + 1.0,
                          0.0)
            inter = iw * ih
            iou = inter / (rarea + garea - inter)
            score = jnp.where(k < nb, iou, -1.0)
            upd = score > mo
            mo = jnp.where(upd, score, mo)
            bk = jnp.where(upd, k, bk)
        mo_ref[b] = mo
        bk_ref[b] = bk
        fgs.append(jnp.where(col_ok, jnp.where(mo >= _FG_THRESH, mo, -1.0),
                             -2.0))
        bgs.append(jnp.where(
            col_ok,
            jnp.where((mo < _BG_HI) & (mo >= _BG_LO), 1.0 - mo, -1.0),
            -2.0))

    def step(i, s, b):
        m = jnp.max(s)
        idx = jnp.min(jnp.where(s == m, lin, big))
        keep_ref[b, i] = idx
        return jnp.where(lin == idx, -3.0, s)

    def phase1(i, c):
        fs = [step(i, c[b], b) for b in range(batch)]
        bs = [step(_FG_ROIS + i, c[batch + b], b) for b in range(batch)]
        return tuple(fs) + tuple(bs)

    c = lax.fori_loop(0, _FG_ROIS, phase1, tuple(fgs) + tuple(bgs))

    def phase2(i, c):
        return tuple(step(_FG_ROIS + i, c[b], b) for b in range(batch))

    lax.fori_loop(_FG_ROIS, _BG_ROIS, phase2, c[batch:])

    colio = lax.broadcasted_iota(jnp.int32, (1, 4 * _NCLASSES), 1)
    cio8 = lax.broadcasted_iota(jnp.int32, (1, 8), 1)
    iota_k = lax.broadcasted_iota(jnp.int32, (1, k_gt), 1)
    rowsel = lax.broadcasted_iota(jnp.int32, (_ROIS, 1), 0)

    for b in range(batch):
        bf = jnp.float32(b)
        rx1 = rois_ref[b, 0]
        ry1 = rois_ref[b, 1]
        rx2 = rois_ref[b, 2]
        ry2 = rois_ref[b, 3]
        mo2 = mo_ref[b]
        bk2 = bk_ref[b]
        gxv1 = gtt_ref[b, pl.ds(0, 1), :]
        gyv1 = gtt_ref[b, pl.ds(1, 1), :]
        gxv2 = gtt_ref[b, pl.ds(2, 1), :]
        gyv2 = gtt_ref[b, pl.ds(3, 1), :]
        gclsv = gtt_ref[b, pl.ds(4, 1), :]

        def emit1(i, accs):
            acc_rois, acc_t, acc_w = accs
            idx = keep_ref[b, i]
            m1 = lin == idx
            ex1 = jnp.sum(jnp.where(m1, rx1, 0.0))
            ey1 = jnp.sum(jnp.where(m1, ry1, 0.0))
            ex2 = jnp.sum(jnp.where(m1, rx2, 0.0))
            ey2 = jnp.sum(jnp.where(m1, ry2, 0.0))
            mo_s = jnp.sum(jnp.where(m1, mo2, 0.0))
            ga = jnp.sum(jnp.where(m1, bk2, 0))
            m2 = iota_k == ga
            gx1 = jnp.sum(jnp.where(m2, gxv1, 0.0))
            gy1 = jnp.sum(jnp.where(m2, gyv1, 0.0))
            gx2 = jnp.sum(jnp.where(m2, gxv2, 0.0))
            gy2 = jnp.sum(jnp.where(m2, gyv2, 0.0))
            gcls = jnp.sum(jnp.where(m2, gclsv, 0.0))

            ew = ex2 - ex1 + 1.0
            eh = ey2 - ey1 + 1.0
            ecx = ex1 + 0.5 * ew
            ecy = ey1 + 0.5 * eh
            gw = gx2 - gx1 + 1.0
            gh = gy2 - gy1 + 1.0
            gcx = gx1 + 0.5 * gw
            gcy = gy1 + 0.5 * gh
            dx = ((gcx - ecx) / ew) / _STDS[0]
            dy = ((gcy - ecy) / eh) / _STDS[1]
            dw = jnp.log(gw / ew) / _STDS[2]
            dh = jnp.log(gh / eh) / _STDS[3]

            fg = mo_s >= _FG_THRESH
            label = jnp.where(fg, gcls, 0.0)

            row = jnp.where(
                cio8 == 0, bf,
                jnp.where(cio8 == 1, ex1,
                          jnp.where(cio8 == 2, ey1,
                                    jnp.where(cio8 == 3, ex2,
                                              jnp.where(cio8 == 4, ey2,
                                                        jnp.where(cio8 == 5,
                                                                  label,
                                                                  0.0))))))

            base = label.astype(jnp.int32) * 4
            off = colio - base
            inr = (off >= 0) & (off < 4) & fg
            tval = jnp.where(off == 0, dx,
                             jnp.where(off == 1, dy,
                                       jnp.where(off == 2, dw, dh)))
            zero = jnp.zeros((1, 4 * _NCLASSES), jnp.float32)
            trow = jnp.where(inr, tval, zero)
            wrow = jnp.where(inr, 1.0, zero)

            here = rowsel == i
            return (jnp.where(here, row, acc_rois),
                    jnp.where(here, trow, acc_t),
                    jnp.where(here, wrow, acc_w))

        def emit4(j, accs):
            for q in range(4):
                accs = emit1(j + 32 * q, accs)
            return accs

        acc0 = (jnp.zeros((_ROIS, 8), jnp.float32),
                jnp.zeros((_ROIS, 4 * _NCLASSES), jnp.float32),
                jnp.zeros((_ROIS, 4 * _NCLASSES), jnp.float32))
        acc_rois, acc_t, acc_w = lax.fori_loop(0, 32, emit4, acc0)
        out_rois[b] = acc_rois
        out_t[b] = acc_t
        out_in[b] = acc_w
        out_out[b] = acc_w


@jax.jit
def kernel(all_rois, gt_boxes, num_boxes):
    B, N, _ = all_rois.shape
    K = gt_boxes.shape[1]
    M = N + K
    MP = ((M + 127) // 128) * 128
    rows = MP // 128

    coords = jnp.concatenate([all_rois[:, :, 1:5], gt_boxes[:, :, :4]], axis=1)
    coords = jnp.pad(coords, ((0, 0), (0, MP - M), (0, 0)))
    rois_t = coords.transpose(0, 2, 1).reshape(B, 4, rows, 128)
    gtt = gt_boxes.transpose(0, 2, 1)
    nb = num_boxes.astype(jnp.int32).reshape(B, 1, 1)

    out_shapes = (
        jax.ShapeDtypeStruct((B, _ROIS, 8), jnp.float32),
        jax.ShapeDtypeStruct((B, _ROIS, 4 * _NCLASSES), jnp.float32),
        jax.ShapeDtypeStruct((B, _ROIS, 4 * _NCLASSES), jnp.float32),
        jax.ShapeDtypeStruct((B, _ROIS, 4 * _NCLASSES), jnp.float32),
    )
    grid_spec = pltpu.PrefetchScalarGridSpec(
        num_scalar_prefetch=0,
        grid=(1,),
        in_specs=[
            pl.BlockSpec((B, 1, 1), lambda i: (0, 0, 0),
                         memory_space=pltpu.SMEM),
            pl.BlockSpec((B, 4, rows, 128), lambda i: (0, 0, 0, 0)),
            pl.BlockSpec((B, 5, K), lambda i: (0, 0, 0)),
        ],
        out_specs=[
            pl.BlockSpec((B, _ROIS, 8), lambda i: (0, 0, 0)),
            pl.BlockSpec((B, _ROIS, 4 * _NCLASSES), lambda i: (0, 0, 0)),
            pl.BlockSpec((B, _ROIS, 4 * _NCLASSES), lambda i: (0, 0, 0)),
            pl.BlockSpec((B, _ROIS, 4 * _NCLASSES), lambda i: (0, 0, 0)),
        ],
        scratch_shapes=[
            pltpu.VMEM((B, rows, 128), jnp.float32),
            pltpu.VMEM((B, rows, 128), jnp.int32),
            pltpu.SMEM((B, _ROIS), jnp.int32),
        ],
    )
    fn = pl.pallas_call(
        functools.partial(_body, M, K, B),
        grid_spec=grid_spec,
        out_shape=out_shapes,
    )
    comb, tgt, win, wout = fn(nb, rois_t, gtt)
    return comb[:, :, :5], comb[:, :, 5], tgt, win, wout


# speedup vs baseline: 2.6951x; 1.7978x over previous
import functools

import jax
import jax.numpy as jnp
from jax import lax
from jax.experimental import pallas as pl
from jax.experimental.pallas import tpu as pltpu

_NCLASSES = 21
_FG_ROIS = 32
_BG_ROIS = 96
_ROIS = _FG_ROIS + _BG_ROIS
_FG_THRESH = 0.5
_BG_HI = 0.5
_BG_LO = 0.1
_STDS = (0.1, 0.1, 0.2, 0.2)
_HI = jax.lax.Precision.HIGHEST


def _body(m_total, k_gt, batch, nb_ref, rois_ref, gtt_ref,
          out_rois, out_t, out_in, out_out):
    rows = rois_ref.shape[2]
    lin = (lax.broadcasted_iota(jnp.int32, (rows, 128), 0) * 128
           + lax.broadcasted_iota(jnp.int32, (rows, 128), 1))
    col_ok = lin < m_total
    big = jnp.int32(rows * 128)
    laneio = lax.broadcasted_iota(jnp.int32, (1, 128), 1)

    fgs = []
    bgs = []
    mos = []
    bks = []
    for b in range(batch):
        nb = nb_ref[b, 0, 0]
        rx1 = rois_ref[b, 0]
        ry1 = rois_ref[b, 1]
        rx2 = rois_ref[b, 2]
        ry2 = rois_ref[b, 3]
        rarea = (rx2 - rx1 + 1.0) * (ry2 - ry1 + 1.0)
        mo = jnp.full((rows, 128), -2.0, jnp.float32)
        bk = jnp.zeros((rows, 128), jnp.int32)
        for k in range(k_gt):
            gx1 = gtt_ref[b, 0, k]
            gy1 = gtt_ref[b, 1, k]
            gx2 = gtt_ref[b, 2, k]
            gy2 = gtt_ref[b, 3, k]
            garea = (gx2 - gx1 + 1.0) * (gy2 - gy1 + 1.0)
            iw = jnp.clip(jnp.minimum(rx2, gx2) - jnp.maximum(rx1, gx1) + 1.0,
                          0.0)
            ih = jnp.clip(jnp.minimum(ry2, gy2) - jnp.maximum(ry1, gy1) + 1.0,
                          0.0)
            inter = iw * ih
            iou = inter / (rarea + garea - inter)
            score = jnp.where(k < nb, iou, -1.0)
            upd = score > mo
            mo = jnp.where(upd, score, mo)
            bk = jnp.where(upd, k, bk)
        mos.append(mo)
        bks.append(bk)
        fgs.append(jnp.where(col_ok, jnp.where(mo >= _FG_THRESH, mo, -1.0),
                             -2.0))
        bgs.append(jnp.where(
            col_ok,
            jnp.where((mo < _BG_HI) & (mo >= _BG_LO), 1.0 - mo, -1.0),
            -2.0))

    def step(pos, s, kv, b):
        m = jnp.max(s)
        idx = jnp.min(jnp.where(s == m, lin, big))
        kv = jnp.where(laneio == pos, idx.astype(jnp.float32), kv)
        return jnp.where(lin == idx, -3.0, s), kv

    kv0 = [jnp.zeros((1, 128), jnp.float32) for _ in range(batch)]

    def phase1(i, c):
        fs = list(c[:batch])
        bs = list(c[batch:2 * batch])
        kvs = list(c[2 * batch:])
        for b in range(batch):
            fs[b], kvs[b] = step(i, fs[b], kvs[b], b)
        for b in range(batch):
            bs[b], kvs[b] = step(_FG_ROIS + i, bs[b], kvs[b], b)
        return tuple(fs) + tuple(bs) + tuple(kvs)

    c = lax.fori_loop(0, _FG_ROIS, phase1,
                      tuple(fgs) + tuple(bgs) + tuple(kv0))

    def phase2(i, c):
        bs = list(c[:batch])
        kvs = list(c[batch:])
        for b in range(batch):
            bs[b], kvs[b] = step(_FG_ROIS + i, bs[b], kvs[b], b)
        return tuple(bs) + tuple(kvs)

    c = lax.fori_loop(_FG_ROIS, _BG_ROIS, phase2, c[batch:])
    kvs = list(c[batch:])

    colio = lax.broadcasted_iota(jnp.int32, (1, 4 * _NCLASSES), 1)
    cio8 = lax.broadcasted_iota(jnp.int32, (1, 8), 1)
    iota_k = lax.broadcasted_iota(jnp.int32, (1, k_gt), 1)
    iota_r = lax.broadcasted_iota(jnp.int32, (1, rows), 1)
    ident = (lax.broadcasted_iota(jnp.int32, (_ROIS, 128), 0)
             == lax.broadcasted_iota(jnp.int32, (_ROIS, 128), 1)
             ).astype(jnp.float32)

    for b in range(batch):
        nb = nb_ref[b, 0, 0]
        bf = jnp.float32(b)
        kcol = lax.dot_general(ident, kvs[b], (((1,), (1,)), ((), ())),
                               precision=_HI)
        kint = kcol.astype(jnp.int32)
        r_col = kint // 128
        c_col = kint - r_col * 128
        rmat = (r_col == iota_r).astype(jnp.float32)
        cmask = c_col == laneio

        planes = jnp.concatenate(
            [rois_ref[b, 0], rois_ref[b, 1], rois_ref[b, 2], rois_ref[b, 3],
             mos[b], bks[b].astype(jnp.float32)], axis=1)
        tmat = lax.dot_general(rmat, planes, (((1,), (0,)), ((), ())),
                               precision=_HI)

        def pick(j):
            tj = tmat[:, 128 * j:128 * (j + 1)]
            return jnp.sum(jnp.where(cmask, tj, 0.0), axis=1, keepdims=True)

        ex1 = pick(0)
        ey1 = pick(1)
        ex2 = pick(2)
        ey2 = pick(3)
        mo_s = pick(4)
        ga = pick(5).astype(jnp.int32)

        gxv1 = gtt_ref[b, pl.ds(0, 1), :]
        gyv1 = gtt_ref[b, pl.ds(1, 1), :]
        gxv2 = gtt_ref[b, pl.ds(2, 1), :]
        gyv2 = gtt_ref[b, pl.ds(3, 1), :]
        gclsv = gtt_ref[b, pl.ds(4, 1), :]
        m2 = iota_k == ga

        def gsel(v):
            return jnp.sum(jnp.where(m2, v, 0.0), axis=1, keepdims=True)

        gx1 = gsel(gxv1)
        gy1 = gsel(gyv1)
        gx2 = gsel(gxv2)
        gy2 = gsel(gyv2)
        gcls = gsel(gclsv)

        ew = ex2 - ex1 + 1.0
        eh = ey2 - ey1 + 1.0
        ecx = ex1 + 0.5 * ew
        ecy = ey1 + 0.5 * eh
        gw = gx2 - gx1 + 1.0
        gh = gy2 - gy1 + 1.0
        gcx = gx1 + 0.5 * gw
        gcy = gy1 + 0.5 * gh
        dx = ((gcx - ecx) / ew) / _STDS[0]
        dy = ((gcy - ecy) / eh) / _STDS[1]
        dw = jnp.log(gw / ew) / _STDS[2]
        dh = jnp.log(gh / eh) / _STDS[3]

        fg = mo_s >= _FG_THRESH
        label = jnp.where(fg, gcls, 0.0)

        comb = jnp.where(
            cio8 == 0, bf,
            jnp.where(cio8 == 1, ex1,
                      jnp.where(cio8 == 2, ey1,
                                jnp.where(cio8 == 3, ex2,
                                          jnp.where(cio8 == 4, ey2,
                                                    jnp.where(cio8 == 5,
                                                              label,
                                                              0.0))))))

        base = label.astype(jnp.int32) * 4
        off = colio - base
        inr = (off >= 0) & (off < 4) & fg
        tval = jnp.where(off == 0, dx,
                         jnp.where(off == 1, dy,
                                   jnp.where(off == 2, dw, dh)))
        trow = jnp.where(inr, tval, 0.0)
        wrow = jnp.where(inr, 1.0, 0.0)

        out_rois[b] = comb
        out_t[b] = trow
        out_in[b] = wrow
        out_out[b] = wrow


@jax.jit
def kernel(all_rois, gt_boxes, num_boxes):
    B, N, _ = all_rois.shape
    K = gt_boxes.shape[1]
    M = N + K
    MP = ((M + 127) // 128) * 128
    rows = MP // 128

    coords = jnp.concatenate([all_rois[:, :, 1:5], gt_boxes[:, :, :4]], axis=1)
    coords = jnp.pad(coords, ((0, 0), (0, MP - M), (0, 0)))
    rois_t = coords.transpose(0, 2, 1).reshape(B, 4, rows, 128)
    gtt = gt_boxes.transpose(0, 2, 1)
    nb = num_boxes.astype(jnp.int32).reshape(B, 1, 1)

    out_shapes = (
        jax.ShapeDtypeStruct((B, _ROIS, 8), jnp.float32),
        jax.ShapeDtypeStruct((B, _ROIS, 4 * _NCLASSES), jnp.float32),
        jax.ShapeDtypeStruct((B, _ROIS, 4 * _NCLASSES), jnp.float32),
        jax.ShapeDtypeStruct((B, _ROIS, 4 * _NCLASSES), jnp.float32),
    )
    grid_spec = pltpu.PrefetchScalarGridSpec(
        num_scalar_prefetch=0,
        grid=(1,),
        in_specs=[
            pl.BlockSpec((B, 1, 1), lambda i: (0, 0, 0),
                         memory_space=pltpu.SMEM),
            pl.BlockSpec((B, 4, rows, 128), lambda i: (0, 0, 0, 0)),
            pl.BlockSpec((B, 5, K), lambda i: (0, 0, 0)),
        ],
        out_specs=[
            pl.BlockSpec((B, _ROIS, 8), lambda i: (0, 0, 0)),
            pl.BlockSpec((B, _ROIS, 4 * _NCLASSES), lambda i: (0, 0, 0)),
            pl.BlockSpec((B, _ROIS, 4 * _NCLASSES), lambda i: (0, 0, 0)),
            pl.BlockSpec((B, _ROIS, 4 * _NCLASSES), lambda i: (0, 0, 0)),
        ],
        scratch_shapes=[],
    )
    fn = pl.pallas_call(
        functools.partial(_body, M, K, B),
        grid_spec=grid_spec,
        out_shape=out_shapes,
    )
    comb, tgt, win, wout = fn(nb, rois_t, gtt)
    return comb[:, :, :5], comb[:, :, 5], tgt, win, wout


# speedup vs baseline: 4.2848x; 1.5898x over previous
import functools

import jax
import jax.numpy as jnp
from jax import lax
from jax.experimental import pallas as pl
from jax.experimental.pallas import tpu as pltpu
from jax.experimental.pallas import tpu_sc as plsc

_NCLASSES = 21
_FG_ROIS = 32
_BG_ROIS = 96
_ROIS = _FG_ROIS + _BG_ROIS
_FG_THRESH = 0.5
_BG_HI = 0.5
_BG_LO = 0.1
_STDS = (0.1, 0.1, 0.2, 0.2)
_HI = jax.lax.Precision.HIGHEST


def _sc_body(m_total, k_gt, batch, chunks,
             coords_hbm, gtt_hbm, nb_hbm, out_hbm,
             cv, mov, bkv, fgv, bgv, cmf, cmb, gtv, nbv, selv, outv):
    lane = lax.broadcasted_iota(jnp.int32, (16,), 0)
    ngrp = chunks // 16

    wid = lax.axis_index("s") * 2 + lax.axis_index("c")

    @pl.when(wid < batch)
    def _():
        b = wid
        pltpu.sync_copy(coords_hbm.at[b], cv)
        pltpu.sync_copy(gtt_hbm.at[b], gtv)
        pltpu.sync_copy(nb_hbm.at[b], nbv)
        nbf = jnp.sum(jnp.where(lane == 0,
                                nbv[pl.ds(0, 16)].astype(jnp.float32), 0.0))

        gts = []
        for k in range(k_gt):
            sl = pl.ds((k // 16) * 16, 16)
            lm = lane == (k % 16)

            def pickg(i, sl=sl, lm=lm):
                return jnp.sum(jnp.where(lm, gtv[i, sl], 0.0))

            gx1 = pickg(0)
            gy1 = pickg(1)
            gx2 = pickg(2)
            gy2 = pickg(3)
            garea = (gx2 - gx1 + 1.0) * (gy2 - gy1 + 1.0)
            gts.append((gx1, gy1, gx2, gy2, garea))

        def chunk_body(c, carry):
            accf, accb = carry
            sl = pl.ds(c * 16, 16)
            rx1 = cv[0, sl]
            ry1 = cv[1, sl]
            rx2 = cv[2, sl]
            ry2 = cv[3, sl]
            ra = (rx2 - rx1 + 1.0) * (ry2 - ry1 + 1.0)
            mo = jnp.full((16,), -2.0, jnp.float32)
            bk = jnp.zeros((16,), jnp.float32)
            for k in range(k_gt):
                gx1, gy1, gx2, gy2, garea = gts[k]
                iw = jnp.maximum(
                    jnp.minimum(rx2, gx2) - jnp.maximum(rx1, gx1) + 1.0, 0.0)
                ih = jnp.maximum(
                    jnp.minimum(ry2, gy2) - jnp.maximum(ry1, gy1) + 1.0, 0.0)
                inter = iw * ih
                iou = inter / (ra + garea - inter)
                score = jnp.where(float(k) < nbf, iou,
                                  jnp.full((16,), -1.0, jnp.float32))
                upd = score > mo
                mo = jnp.where(upd, score, mo)
                bk = jnp.where(upd, float(k), bk)
            gidx = c * 16 + lane
            colok = gidx < m_total
            fg = jnp.where(colok,
                           jnp.where(mo >= _FG_THRESH, mo, -1.0), -2.0)
            bg = jnp.where(
                colok,
                jnp.where((mo < _BG_HI) & (mo >= _BG_LO), 1.0 - mo, -1.0),
                -2.0)
            mov[sl] = mo
            bkv[sl] = bk
            fgv[sl] = fg
            bgv[sl] = bg
            q = c % 16
            accf = jnp.where(lane == q, jnp.max(fg), accf)
            accb = jnp.where(lane == q, jnp.max(bg), accb)

            @pl.when(q == 15)
            def _():
                g = c // 16
                cmf[pl.ds(g * 16, 16)] = accf
                cmb[pl.ds(g * 16, 16)] = accb

            return accf, accb

        z16 = jnp.zeros((16,), jnp.float32)
        lax.fori_loop(0, chunks, chunk_body, (z16, z16))

        def extract(sv, cm, start, count):
            def step(t, acc):
                av = jnp.full((16,), -3.4e38, jnp.float32)
                for i in range(ngrp):
                    av = jnp.maximum(av, cm[pl.ds(i * 16, 16)])
                m = jnp.max(av)
                iv = jnp.full((16,), jnp.int32(10 ** 6), jnp.int32)
                for i in range(ngrp):
                    ch = cm[pl.ds(i * 16, 16)]
                    iv = jnp.minimum(
                        iv, jnp.where(ch == m, i * 16 + lane, 10 ** 6))
                cid = jnp.min(iv)
                so = cid * 16
                ch = sv[pl.ds(so, 16)]
                ln = jnp.min(jnp.where(ch == m, lane, 16))
                gidx = so + ln
                ch2 = jnp.where(lane == ln, -3.0, ch)
                sv[pl.ds(so, 16)] = ch2
                nm = jnp.max(ch2)
                s = cid // 16
                q2 = cid - s * 16
                cs = cm[pl.ds(s * 16, 16)]
                cm[pl.ds(s * 16, 16)] = jnp.where(lane == q2, nm, cs)
                acc = jnp.where(lane == (t % 16), gidx, acc)

                @pl.when((t % 16) == 15)
                def _():
                    selv[pl.ds(start + (t // 16) * 16, 16)] = acc

                return acc

            lax.fori_loop(0, count, step, jnp.zeros((16,), jnp.int32),
                          unroll=False)

        extract(fgv, cmf, 0, _FG_ROIS)
        extract(bgv, cmb, _FG_ROIS, _BG_ROIS)

        zi = jnp.zeros((16,), jnp.int32)
        for g in range(_ROIS // 16):
            sl = pl.ds(g * 16, 16)
            idx = selv[sl]
            outv[0, sl] = plsc.load_gather(cv, [zi, idx])
            outv[1, sl] = plsc.load_gather(cv, [zi + 1, idx])
            outv[2, sl] = plsc.load_gather(cv, [zi + 2, idx])
            outv[3, sl] = plsc.load_gather(cv, [zi + 3, idx])
            outv[4, sl] = plsc.load_gather(mov, [idx])
            outv[5, sl] = plsc.load_gather(bkv, [idx])
        pltpu.sync_copy(outv, out_hbm.at[b])


def _tc_body(k_gt, batch, sel_ref, gtt_ref, out_rois, out_t, out_in, out_out):
    laneio = lax.broadcasted_iota(jnp.int32, (1, 128), 1)
    colio = lax.broadcasted_iota(jnp.int32, (1, 4 * _NCLASSES), 1)
    cio8 = lax.broadcasted_iota(jnp.int32, (1, 8), 1)
    iota_k = lax.broadcasted_iota(jnp.int32, (1, k_gt), 1)
    ident = (lax.broadcasted_iota(jnp.int32, (_ROIS, 128), 0)
             == lax.broadcasted_iota(jnp.int32, (_ROIS, 128), 1)
             ).astype(jnp.float32)
    del laneio

    for b in range(batch):
        bf = jnp.float32(b)

        def tr(i, b=b):
            row = sel_ref[b, pl.ds(i, 1), :]
            return lax.dot_general(ident, row, (((1,), (1,)), ((), ())),
                                   precision=_HI)

        ex1 = tr(0)
        ey1 = tr(1)
        ex2 = tr(2)
        ey2 = tr(3)
        mo_s = tr(4)
        ga = tr(5).astype(jnp.int32)

        gxv1 = gtt_ref[b, pl.ds(0, 1), :]
        gyv1 = gtt_ref[b, pl.ds(1, 1), :]
        gxv2 = gtt_ref[b, pl.ds(2, 1), :]
        gyv2 = gtt_ref[b, pl.ds(3, 1), :]
        gclsv = gtt_ref[b, pl.ds(4, 1), :]
        m2 = iota_k == ga

        def gsel(v):
            return jnp.sum(jnp.where(m2, v, 0.0), axis=1, keepdims=True)

        gx1 = gsel(gxv1)
        gy1 = gsel(gyv1)
        gx2 = gsel(gxv2)
        gy2 = gsel(gyv2)
        gcls = gsel(gclsv)

        ew = ex2 - ex1 + 1.0
        eh = ey2 - ey1 + 1.0
        ecx = ex1 + 0.5 * ew
        ecy = ey1 + 0.5 * eh
        gw = gx2 - gx1 + 1.0
        gh = gy2 - gy1 + 1.0
        gcx = gx1 + 0.5 * gw
        gcy = gy1 + 0.5 * gh
        dx = ((gcx - ecx) / ew) / _STDS[0]
        dy = ((gcy - ecy) / eh) / _STDS[1]
        dw = jnp.log(gw / ew) / _STDS[2]
        dh = jnp.log(gh / eh) / _STDS[3]

        fg = mo_s >= _FG_THRESH
        label = jnp.where(fg, gcls, 0.0)

        comb = jnp.where(
            cio8 == 0, bf,
            jnp.where(cio8 == 1, ex1,
                      jnp.where(cio8 == 2, ey1,
                                jnp.where(cio8 == 3, ex2,
                                          jnp.where(cio8 == 4, ey2,
                                                    jnp.where(cio8 == 5,
                                                              label,
                                                              0.0))))))

        base = label.astype(jnp.int32) * 4
        off = colio - base
        inr = (off >= 0) & (off < 4) & fg
        tval = jnp.where(off == 0, dx,
                         jnp.where(off == 1, dy,
                                   jnp.where(off == 2, dw, dh)))
        trow = jnp.where(inr, tval, 0.0)
        wrow = jnp.where(inr, 1.0, 0.0)

        out_rois[b] = comb
        out_t[b] = trow
        out_in[b] = wrow
        out_out[b] = wrow


@jax.jit
def kernel(all_rois, gt_boxes, num_boxes):
    B, N, _ = all_rois.shape
    K = gt_boxes.shape[1]
    M = N + K
    MP = ((M + 127) // 128) * 128
    chunks = MP // 16

    coords = jnp.concatenate([all_rois[:, :, 1:5], gt_boxes[:, :, :4]], axis=1)
    coords = jnp.pad(coords, ((0, 0), (0, MP - M), (0, 0)))
    coords_t = coords.transpose(0, 2, 1)
    gtt = gt_boxes.transpose(0, 2, 1)
    gtp = jnp.pad(gtt, ((0, 0), (0, 0), (0, 32 - K)))
    nb16 = jnp.pad(num_boxes.astype(jnp.int32)[:, None],
                   ((0, 0), (0, 15)))

    mesh = plsc.VectorSubcoreMesh(core_axis_name="c", subcore_axis_name="s")
    sc_fn = functools.partial(
        pl.kernel,
        mesh=mesh,
        compiler_params=pltpu.CompilerParams(needs_layout_passes=False),
        out_type=jax.ShapeDtypeStruct((B, 6, 128), jnp.float32),
        scratch_types=[
            pltpu.VMEM((4, MP), jnp.float32),
            pltpu.VMEM((MP,), jnp.float32),
            pltpu.VMEM((MP,), jnp.float32),
            pltpu.VMEM((MP,), jnp.float32),
            pltpu.VMEM((MP,), jnp.float32),
            pltpu.VMEM((chunks,), jnp.float32),
            pltpu.VMEM((chunks,), jnp.float32),
            pltpu.VMEM((5, 32), jnp.float32),
            pltpu.VMEM((16,), jnp.int32),
            pltpu.VMEM((_ROIS,), jnp.int32),
            pltpu.VMEM((6, 128), jnp.float32),
        ],
    )(functools.partial(_sc_body, M, K, B, chunks))
    sel = sc_fn(coords_t, gtp, nb16)

    out_shapes = (
        jax.ShapeDtypeStruct((B, _ROIS, 8), jnp.float32),
        jax.ShapeDtypeStruct((B, _ROIS, 4 * _NCLASSES), jnp.float32),
        jax.ShapeDtypeStruct((B, _ROIS, 4 * _NCLASSES), jnp.float32),
        jax.ShapeDtypeStruct((B, _ROIS, 4 * _NCLASSES), jnp.float32),
    )
    grid_spec = pltpu.PrefetchScalarGridSpec(
        num_scalar_prefetch=0,
        grid=(1,),
        in_specs=[
            pl.BlockSpec((B, 6, 128), lambda i: (0, 0, 0)),
            pl.BlockSpec((B, 5, K), lambda i: (0, 0, 0)),
        ],
        out_specs=[
            pl.BlockSpec((B, _ROIS, 8), lambda i: (0, 0, 0)),
            pl.BlockSpec((B, _ROIS, 4 * _NCLASSES), lambda i: (0, 0, 0)),
            pl.BlockSpec((B, _ROIS, 4 * _NCLASSES), lambda i: (0, 0, 0)),
            pl.BlockSpec((B, _ROIS, 4 * _NCLASSES), lambda i: (0, 0, 0)),
        ],
        scratch_shapes=[],
    )
    fn = pl.pallas_call(
        functools.partial(_tc_body, K, B),
        grid_spec=grid_spec,
        out_shape=out_shapes,
    )
    comb, tgt, win, wout = fn(sel, gtt)
    return comb[:, :, :5], comb[:, :, 5], tgt, win, wout


# speedup vs baseline: 5.7082x; 1.3322x over previous
import functools

import jax
import jax.numpy as jnp
from jax import lax
from jax.experimental import pallas as pl
from jax.experimental.pallas import tpu as pltpu
from jax.experimental.pallas import tpu_sc as plsc

_NCLASSES = 21
_FG_ROIS = 32
_BG_ROIS = 96
_ROIS = _FG_ROIS + _BG_ROIS
_FG_THRESH = 0.5
_BG_HI = 0.5
_BG_LO = 0.1
_STDS = (0.1, 0.1, 0.2, 0.2)
_HI = jax.lax.Precision.HIGHEST


def _sc_body(m_total, k_gt, batch, chunks,
             coords_hbm, gtt_hbm, nb_hbm, out_hbm,
             cv, pv, cmf, cmb, gtv, nbv, selv, outv, sh):
    lane = lax.broadcasted_iota(jnp.int32, (16,), 0)
    ngrp = chunks // 16
    pchunks = chunks // 8

    c_ax = lax.axis_index("c")
    s_ax = lax.axis_index("s")
    li = s_ax // 8
    part = s_ax % 8
    img = c_ax * 2 + li

    pltpu.sync_copy(coords_hbm.at[img], cv)
    pltpu.sync_copy(gtt_hbm.at[img], gtv)
    pltpu.sync_copy(nb_hbm.at[img], nbv)
    nbf = jnp.sum(jnp.where(lane == 0,
                            nbv[pl.ds(0, 16)].astype(jnp.float32), 0.0))

    gts = []
    for k in range(k_gt):
        ksl = pl.ds((k // 16) * 16, 16)
        lm = lane == (k % 16)

        def pickg(i, ksl=ksl, lm=lm):
            return jnp.sum(jnp.where(lm, gtv[i, ksl], 0.0))

        gx1 = pickg(0)
        gy1 = pickg(1)
        gx2 = pickg(2)
        gy2 = pickg(3)
        garea = (gx2 - gx1 + 1.0) * (gy2 - gy1 + 1.0)
        gts.append((gx1, gy1, gx2, gy2, garea))

    def chunk_body(c, carry):
        sl = pl.ds(c * 16, 16)
        rx1 = cv[0, sl]
        ry1 = cv[1, sl]
        rx2 = cv[2, sl]
        ry2 = cv[3, sl]
        ra = (rx2 - rx1 + 1.0) * (ry2 - ry1 + 1.0)
        mo = jnp.full((16,), -2.0, jnp.float32)
        bk = jnp.zeros((16,), jnp.float32)
        for k in range(k_gt):
            gx1, gy1, gx2, gy2, garea = gts[k]
            iw = jnp.maximum(
                jnp.minimum(rx2, gx2) - jnp.maximum(rx1, gx1) + 1.0, 0.0)
            ih = jnp.maximum(
                jnp.minimum(ry2, gy2) - jnp.maximum(ry1, gy1) + 1.0, 0.0)
            inter = iw * ih
            iou = inter / (ra + garea - inter)
            score = jnp.where(float(k) < nbf, iou,
                              jnp.full((16,), -1.0, jnp.float32))
            upd = score > mo
            mo = jnp.where(upd, score, mo)
            bk = jnp.where(upd, float(k), bk)
        gidx = c * 16 + lane
        colok = gidx < m_total
        pv[0, sl] = jnp.where(
            colok, jnp.where(mo >= _FG_THRESH, mo, -1.0), -2.0)
        pv[1, sl] = jnp.where(
            colok,
            jnp.where((mo < _BG_HI) & (mo >= _BG_LO), 1.0 - mo, -1.0),
            -2.0)
        pv[2, sl] = mo
        pv[3, sl] = bk
        return carry

    lax.fori_loop(part * pchunks, (part + 1) * pchunks, chunk_body, 0)
    psl = pl.ds(part * pchunks * 16, pchunks * 16)
    pltpu.sync_copy(pv.at[:, psl], sh.at[li, :, psl])
    plsc.subcore_barrier()

    @pl.when(part == 0)
    def _():
        pltpu.sync_copy(sh.at[li], pv)

        z16 = jnp.zeros((16,), jnp.float32)

        def grp(g, carry):
            def inner(j, acc):
                accf, accb = acc
                sl = pl.ds((g * 16 + j) * 16, 16)
                accf = jnp.where(lane == j, jnp.max(pv[0, sl]), accf)
                accb = jnp.where(lane == j, jnp.max(pv[1, sl]), accb)
                return accf, accb

            accf, accb = lax.fori_loop(0, 16, inner, (z16, z16))
            cmf[pl.ds(g * 16, 16)] = accf
            cmb[pl.ds(g * 16, 16)] = accb
            return carry

        lax.fori_loop(0, ngrp, grp, 0)

        def extract(row, cm, start, count):
            def step(t, acc):
                av = jnp.full((16,), -3.4e38, jnp.float32)
                for i in range(ngrp):
                    av = jnp.maximum(av, cm[pl.ds(i * 16, 16)])
                m = jnp.max(av)
                iv = jnp.full((16,), jnp.int32(10 ** 6), jnp.int32)
                for i in range(ngrp):
                    ch = cm[pl.ds(i * 16, 16)]
                    iv = jnp.minimum(
                        iv, jnp.where(ch == m, i * 16 + lane, 10 ** 6))
                cid = jnp.min(iv)
                so = cid * 16
                ch = pv[row, pl.ds(so, 16)]
                ln = jnp.min(jnp.where(ch == m, lane, 16))
                gidx = so + ln
                ch2 = jnp.where(lane == ln, -3.0, ch)
                pv[row, pl.ds(so, 16)] = ch2
                nm = jnp.max(ch2)
                s = cid // 16
                q2 = cid - s * 16
                cs = cm[pl.ds(s * 16, 16)]
                cm[pl.ds(s * 16, 16)] = jnp.where(lane == q2, nm, cs)
                acc = jnp.where(lane == (t % 16), gidx, acc)

                @pl.when((t % 16) == 15)
                def _():
                    selv[pl.ds(start + (t // 16) * 16, 16)] = acc

                return acc

            lax.fori_loop(0, count, step, jnp.zeros((16,), jnp.int32),
                          unroll=False)

        extract(0, cmf, 0, _FG_ROIS)
        extract(1, cmb, _FG_ROIS, _BG_ROIS)

        zi = jnp.zeros((16,), jnp.int32)
        for g in range(_ROIS // 16):
            sl = pl.ds(g * 16, 16)
            idx = selv[sl]
            outv[0, sl] = plsc.load_gather(cv, [zi, idx])
            outv[1, sl] = plsc.load_gather(cv, [zi + 1, idx])
            outv[2, sl] = plsc.load_gather(cv, [zi + 2, idx])
            outv[3, sl] = plsc.load_gather(cv, [zi + 3, idx])
            outv[4, sl] = plsc.load_gather(pv, [zi + 2, idx])
            outv[5, sl] = plsc.load_gather(pv, [zi + 3, idx])
        pltpu.sync_copy(outv, out_hbm.at[img])


def _tc_body(k_gt, batch, sel_ref, gtt_ref, out_rois, out_t, out_in, out_out):
    laneio = lax.broadcasted_iota(jnp.int32, (1, 128), 1)
    colio = lax.broadcasted_iota(jnp.int32, (1, 4 * _NCLASSES), 1)
    cio8 = lax.broadcasted_iota(jnp.int32, (1, 8), 1)
    iota_k = lax.broadcasted_iota(jnp.int32, (1, k_gt), 1)
    ident = (lax.broadcasted_iota(jnp.int32, (_ROIS, 128), 0)
             == lax.broadcasted_iota(jnp.int32, (_ROIS, 128), 1)
             ).astype(jnp.float32)
    del laneio

    for b in range(batch):
        bf = jnp.float32(b)

        def tr(i, b=b):
            row = sel_ref[b, pl.ds(i, 1), :]
            return lax.dot_general(ident, row, (((1,), (1,)), ((), ())),
                                   precision=_HI)

        ex1 = tr(0)
        ey1 = tr(1)
        ex2 = tr(2)
        ey2 = tr(3)
        mo_s = tr(4)
        ga = tr(5).astype(jnp.int32)

        gxv1 = gtt_ref[b, pl.ds(0, 1), :]
        gyv1 = gtt_ref[b, pl.ds(1, 1), :]
        gxv2 = gtt_ref[b, pl.ds(2, 1), :]
        gyv2 = gtt_ref[b, pl.ds(3, 1), :]
        gclsv = gtt_ref[b, pl.ds(4, 1), :]
        m2 = iota_k == ga

        def gsel(v):
            return jnp.sum(jnp.where(m2, v, 0.0), axis=1, keepdims=True)

        gx1 = gsel(gxv1)
        gy1 = gsel(gyv1)
        gx2 = gsel(gxv2)
        gy2 = gsel(gyv2)
        gcls = gsel(gclsv)

        ew = ex2 - ex1 + 1.0
        eh = ey2 - ey1 + 1.0
        ecx = ex1 + 0.5 * ew
        ecy = ey1 + 0.5 * eh
        gw = gx2 - gx1 + 1.0
        gh = gy2 - gy1 + 1.0
        gcx = gx1 + 0.5 * gw
        gcy = gy1 + 0.5 * gh
        dx = ((gcx - ecx) / ew) / _STDS[0]
        dy = ((gcy - ecy) / eh) / _STDS[1]
        dw = jnp.log(gw / ew) / _STDS[2]
        dh = jnp.log(gh / eh) / _STDS[3]

        fg = mo_s >= _FG_THRESH
        label = jnp.where(fg, gcls, 0.0)

        comb = jnp.where(
            cio8 == 0, bf,
            jnp.where(cio8 == 1, ex1,
                      jnp.where(cio8 == 2, ey1,
                                jnp.where(cio8 == 3, ex2,
                                          jnp.where(cio8 == 4, ey2,
                                                    jnp.where(cio8 == 5,
                                                              label,
                                                              0.0))))))

        base = label.astype(jnp.int32) * 4
        off = colio - base
        inr = (off >= 0) & (off < 4) & fg
        tval = jnp.where(off == 0, dx,
                         jnp.where(off == 1, dy,
                                   jnp.where(off == 2, dw, dh)))
        trow = jnp.where(inr, tval, 0.0)
        wrow = jnp.where(inr, 1.0, 0.0)

        out_rois[b] = comb
        out_t[b] = trow
        out_in[b] = wrow
        out_out[b] = wrow


@jax.jit
def kernel(all_rois, gt_boxes, num_boxes):
    B, N, _ = all_rois.shape
    K = gt_boxes.shape[1]
    M = N + K
    MP = ((M + 127) // 128) * 128
    chunks = MP // 16

    coords = jnp.concatenate([all_rois[:, :, 1:5], gt_boxes[:, :, :4]], axis=1)
    coords = jnp.pad(coords, ((0, 0), (0, MP - M), (0, 0)))
    coords_t = coords.transpose(0, 2, 1)
    gtt = gt_boxes.transpose(0, 2, 1)
    gtp = jnp.pad(gtt, ((0, 0), (0, 0), (0, 32 - K)))
    nb16 = jnp.pad(num_boxes.astype(jnp.int32)[:, None],
                   ((0, 0), (0, 15)))

    mesh = plsc.VectorSubcoreMesh(core_axis_name="c", subcore_axis_name="s")
    sc_fn = functools.partial(
        pl.kernel,
        mesh=mesh,
        compiler_params=pltpu.CompilerParams(needs_layout_passes=False),
        out_type=jax.ShapeDtypeStruct((B, 6, 128), jnp.float32),
        scratch_types=[
            pltpu.VMEM((4, MP), jnp.float32),
            pltpu.VMEM((4, MP), jnp.float32),
            pltpu.VMEM((chunks,), jnp.float32),
            pltpu.VMEM((chunks,), jnp.float32),
            pltpu.VMEM((5, 32), jnp.float32),
            pltpu.VMEM((16,), jnp.int32),
            pltpu.VMEM((_ROIS,), jnp.int32),
            pltpu.VMEM((6, 128), jnp.float32),
            pltpu.VMEM_SHARED((2, 4, MP), jnp.float32),
        ],
    )(functools.partial(_sc_body, M, K, B, chunks))
    sel = sc_fn(coords_t, gtp, nb16)

    out_shapes = (
        jax.ShapeDtypeStruct((B, _ROIS, 8), jnp.float32),
        jax.ShapeDtypeStruct((B, _ROIS, 4 * _NCLASSES), jnp.float32),
        jax.ShapeDtypeStruct((B, _ROIS, 4 * _NCLASSES), jnp.float32),
        jax.ShapeDtypeStruct((B, _ROIS, 4 * _NCLASSES), jnp.float32),
    )
    grid_spec = pltpu.PrefetchScalarGridSpec(
        num_scalar_prefetch=0,
        grid=(1,),
        in_specs=[
            pl.BlockSpec((B, 6, 128), lambda i: (0, 0, 0)),
            pl.BlockSpec((B, 5, K), lambda i: (0, 0, 0)),
        ],
        out_specs=[
            pl.BlockSpec((B, _ROIS, 8), lambda i: (0, 0, 0)),
            pl.BlockSpec((B, _ROIS, 4 * _NCLASSES), lambda i: (0, 0, 0)),
            pl.BlockSpec((B, _ROIS, 4 * _NCLASSES), lambda i: (0, 0, 0)),
            pl.BlockSpec((B, _ROIS, 4 * _NCLASSES), lambda i: (0, 0, 0)),
        ],
        scratch_shapes=[],
    )
    fn = pl.pallas_call(
        functools.partial(_tc_body, K, B),
        grid_spec=grid_spec,
        out_shape=out_shapes,
    )
    comb, tgt, win, wout = fn(sel, gtt)
    return comb[:, :, :5], comb[:, :, 5], tgt, win, wout
